# baseline (speedup 1.0000x reference)
import jax
import jax.numpy as jnp
from jax import lax
from jax.experimental import pallas as pl
from jax.experimental.pallas import tpu as pltpu
from jax.experimental.pallas import tpu_sc as plsc

_B = 128
_N = 8192
_K = 32
_L = 16
_NC = 2
_NW = 32
_RPW = _B // _NW
_ROUNDS = 8
_PICKS = 4
_BIG = 2 ** 30
_IN_BOUNDS = lax.GatherScatterMode.PROMISE_IN_BOUNDS


def _gat(vec, idx):
    return vec.at[idx].get(mode=_IN_BOUNDS)


def _sc_body(x_hbm, idx_hbm, dist_hbm, row_v, outs_i, outs_d):
    wid = lax.axis_index("s") * _NC + lax.axis_index("c")
    base = wid * _RPW
    pltpu.sync_copy(x_hbm.at[pl.ds(base, _RPW)], row_v)
    lanes = lax.broadcasted_iota(jnp.int32, (_L,), 0)
    inf_v = jnp.full((_L,), jnp.inf, jnp.float32)
    big_v = jnp.full((_L,), _BIG, jnp.int32)

    def row_body(r, dummy):
        mprev = jnp.full((_L,), -jnp.inf, jnp.float32)
        iprev = jnp.full((_L,), -1, jnp.int32)
        out_d = [inf_v, inf_v]
        out_i = [big_v, big_v]
        for rnd in range(_ROUNDS):
            def scan_body(g, carry, mprev=mprev, iprev=iprev):
                bs = list(carry[:_PICKS])
                js = list(carry[_PICKS:])
                for u in range(4):
                    off = g * 64 + u * _L
                    v = row_v[r, pl.ds(off, _L)]
                    iv = lanes + off
                    ok = (v > mprev) | ((v == mprev) & (iv > iprev))
                    w = jnp.where(ok, v, jnp.float32(jnp.inf))
                    iw = iv
                    for t in range(_PICKS):
                        c = w < bs[t]
                        nb = jnp.where(c, w, bs[t])
                        nj = jnp.where(c, iw, js[t])
                        if t + 1 < _PICKS:
                            w = jnp.where(c, bs[t], w)
                            iw = jnp.where(c, js[t], iw)
                        bs[t] = nb
                        js[t] = nj
                return tuple(bs) + tuple(js)

            init = (inf_v,) * _PICKS + (big_v,) * _PICKS
            res = lax.fori_loop(0, _N // 64, scan_body, init)
            bs = list(res[:_PICKS])
            js = list(res[_PICKS:])
            for t in range(_PICKS):
                wv, wi = bs[0], js[0]
                for sh in (1, 2, 4, 8):
                    ov = _gat(wv, lanes ^ sh)
                    oi = _gat(wi, lanes ^ sh)
                    c = (ov < wv) | ((ov == wv) & (oi < wi))
                    wv = jnp.where(c, ov, wv)
                    wi = jnp.where(c, oi, wi)
                k = rnd * _PICKS + t
                half, slot = divmod(k, _L)
                sel = lanes == slot
                out_d[half] = jnp.where(sel, wv, out_d[half])
                out_i[half] = jnp.where(sel, wi, out_i[half])
                m = js[0] == wi
                for t2 in range(_PICKS - 1):
                    bs[t2] = jnp.where(m, bs[t2 + 1], bs[t2])
                    js[t2] = jnp.where(m, js[t2 + 1], js[t2])
                bs[_PICKS - 1] = jnp.where(m, inf_v, bs[_PICKS - 1])
                js[_PICKS - 1] = jnp.where(m, big_v, js[_PICKS - 1])
                if t == _PICKS - 1:
                    mprev, iprev = wv, wi
        outs_d[0, 0:_L] = out_d[0]
        outs_d[0, _L:_K] = out_d[1]
        outs_i[0, 0:_L] = out_i[0]
        outs_i[0, _L:_K] = out_i[1]
        pltpu.sync_copy(outs_i, idx_hbm.at[pl.ds(base + r, 1)])
        pltpu.sync_copy(outs_d, dist_hbm.at[pl.ds(base + r, 1)])
        return dummy

    lax.fori_loop(0, _RPW, row_body, 0)


def kernel(dist_pot_donors, n_neighbors):
    del n_neighbors
    mesh = plsc.VectorSubcoreMesh(core_axis_name="c", subcore_axis_name="s")
    idx, dist = pl.kernel(
        _sc_body,
        out_type=[
            jax.ShapeDtypeStruct((_B, _K), jnp.int32),
            jax.ShapeDtypeStruct((_B, _K), jnp.float32),
        ],
        mesh=mesh,
        scratch_types=[
            pltpu.VMEM((_RPW, _N), jnp.float32),
            pltpu.VMEM((1, _K), jnp.int32),
            pltpu.VMEM((1, _K), jnp.float32),
        ],
    )(dist_pot_donors)
    return (idx, dist)


# speedup vs baseline: 1.9931x; 1.9931x over previous
import jax
import jax.numpy as jnp
from jax import lax
from jax.experimental import pallas as pl
from jax.experimental.pallas import tpu as pltpu
from jax.experimental.pallas import tpu_sc as plsc

_BSC = 32
_N = 8192
_K = 32
_L = 16
_NC = 2
_NW = 32
_RPW = _BSC // _NW
_ROUNDS = 8
_PICKS = 4
_BIG = 2 ** 30
_IN_BOUNDS = lax.GatherScatterMode.PROMISE_IN_BOUNDS


def _gat(vec, idx):
    return vec.at[idx].get(mode=_IN_BOUNDS)


def _sc_body(x_hbm, idx_hbm, dist_hbm, row_v, outs_i, outs_d):
    wid = lax.axis_index("s") * _NC + lax.axis_index("c")
    base = wid * _RPW
    pltpu.sync_copy(x_hbm.at[pl.ds(base, _RPW)], row_v)
    lanes = lax.broadcasted_iota(jnp.int32, (_L,), 0)
    inf_v = jnp.full((_L,), jnp.inf, jnp.float32)
    big_v = jnp.full((_L,), _BIG, jnp.int32)

    def row_body(r, dummy):
        mprev = jnp.full((_L,), -jnp.inf, jnp.float32)
        iprev = jnp.full((_L,), -1, jnp.int32)
        out_d = [inf_v, inf_v]
        out_i = [big_v, big_v]
        for rnd in range(_ROUNDS):
            def scan_body(g, carry, mprev=mprev, iprev=iprev):
                bs = list(carry[:_PICKS])
                js = list(carry[_PICKS:])
                for u in range(4):
                    off = g * 64 + u * _L
                    v = row_v[r, pl.ds(off, _L)]
                    iv = lanes + off
                    ok = (v > mprev) | ((v == mprev) & (iv > iprev))
                    w = jnp.where(ok, v, jnp.float32(jnp.inf))
                    iw = iv
                    for t in range(_PICKS):
                        c = w < bs[t]
                        nb = jnp.where(c, w, bs[t])
                        nj = jnp.where(c, iw, js[t])
                        if t + 1 < _PICKS:
                            w = jnp.where(c, bs[t], w)
                            iw = jnp.where(c, js[t], iw)
                        bs[t] = nb
                        js[t] = nj
                return tuple(bs) + tuple(js)

            init = (inf_v,) * _PICKS + (big_v,) * _PICKS
            res = lax.fori_loop(0, _N // 64, scan_body, init)
            bs = list(res[:_PICKS])
            js = list(res[_PICKS:])
            for t in range(_PICKS):
                wv, wi = bs[0], js[0]
                for sh in (1, 2, 4, 8):
                    ov = _gat(wv, lanes ^ sh)
                    oi = _gat(wi, lanes ^ sh)
                    c = (ov < wv) | ((ov == wv) & (oi < wi))
                    wv = jnp.where(c, ov, wv)
                    wi = jnp.where(c, oi, wi)
                k = rnd * _PICKS + t
                half, slot = divmod(k, _L)
                sel = lanes == slot
                out_d[half] = jnp.where(sel, wv, out_d[half])
                out_i[half] = jnp.where(sel, wi, out_i[half])
                m = js[0] == wi
                for t2 in range(_PICKS - 1):
                    bs[t2] = jnp.where(m, bs[t2 + 1], bs[t2])
                    js[t2] = jnp.where(m, js[t2 + 1], js[t2])
                bs[_PICKS - 1] = jnp.where(m, inf_v, bs[_PICKS - 1])
                js[_PICKS - 1] = jnp.where(m, big_v, js[_PICKS - 1])
                if t == _PICKS - 1:
                    mprev, iprev = wv, wi
        outs_d[0, 0:_L] = out_d[0]
        outs_d[0, _L:_K] = out_d[1]
        outs_i[0, 0:_L] = out_i[0]
        outs_i[0, _L:_K] = out_i[1]
        pltpu.sync_copy(outs_i, idx_hbm.at[pl.ds(base + r, 1)])
        pltpu.sync_copy(outs_d, dist_hbm.at[pl.ds(base + r, 1)])
        return dummy

    lax.fori_loop(0, _RPW, row_body, 0)


def _sc_topk(x):
    mesh = plsc.VectorSubcoreMesh(core_axis_name="c", subcore_axis_name="s")
    idx, dist = pl.kernel(
        _sc_body,
        out_type=[
            jax.ShapeDtypeStruct((_BSC, _K), jnp.int32),
            jax.ShapeDtypeStruct((_BSC, _K), jnp.float32),
        ],
        mesh=mesh,
        scratch_types=[
            pltpu.VMEM((_RPW, _N), jnp.float32),
            pltpu.VMEM((1, _K), jnp.int32),
            pltpu.VMEM((1, _K), jnp.float32),
        ],
    )(x)
    return idx, dist


_B = 128
_BTC = _B - _BSC


def _tc_body(x_ref, idx_ref, dist_ref, work_ref):
    work_ref[:] = x_ref[:]
    cols = lax.broadcasted_iota(jnp.int32, (_BTC, _N), 1)
    for k in range(_K):
        x = work_ref[:]
        m = jnp.min(x, axis=1, keepdims=True)
        hit = x == m
        idx = jnp.min(jnp.where(hit, cols, jnp.int32(_N)), axis=1,
                      keepdims=True)
        dist_ref[:, k] = m[:, 0]
        idx_ref[:, k] = idx[:, 0]
        if k + 1 < _K:
            work_ref[:] = jnp.where(cols == idx, jnp.float32(jnp.inf), x)


def _tc_topk(x):
    idx, dist = pl.pallas_call(
        _tc_body,
        out_shape=[
            jax.ShapeDtypeStruct((_BTC, _K), jnp.int32),
            jax.ShapeDtypeStruct((_BTC, _K), jnp.float32),
        ],
        scratch_shapes=[pltpu.VMEM((_BTC, _N), jnp.float32)],
    )(x)
    return idx, dist


def kernel(dist_pot_donors, n_neighbors):
    del n_neighbors
    sc_idx, sc_dist = _sc_topk(dist_pot_donors[_BTC:])
    tc_idx, tc_dist = _tc_topk(dist_pot_donors[:_BTC])
    return (jnp.concatenate([tc_idx, sc_idx], axis=0),
            jnp.concatenate([tc_dist, sc_dist], axis=0))


# speedup vs baseline: 2.6243x; 1.3167x over previous
import jax
import jax.numpy as jnp
from jax.experimental import pallas as pl
from jax.experimental.pallas import tpu as pltpu

_B = 128
_N = 8192
_K = 32


def _topk_body(x_ref, idx_ref, dist_ref, work_ref):
    work_ref[:] = x_ref[:]
    cols = jax.lax.broadcasted_iota(jnp.int32, (_B, _N), 1)
    for k in range(_K):
        x = work_ref[:]
        m = jnp.min(x, axis=1, keepdims=True)
        hit = x == m
        idx = jnp.min(jnp.where(hit, cols, jnp.int32(_N)), axis=1, keepdims=True)
        dist_ref[:, k] = m[:, 0]
        idx_ref[:, k] = idx[:, 0]
        if k + 1 < _K:
            work_ref[:] = jnp.where(cols == idx, jnp.float32(jnp.inf), x)


def kernel(dist_pot_donors, n_neighbors):
    del n_neighbors
    idx, dist = pl.pallas_call(
        _topk_body,
        out_shape=[
            jax.ShapeDtypeStruct((_B, _K), jnp.int32),
            jax.ShapeDtypeStruct((_B, _K), jnp.float32),
        ],
        scratch_shapes=[pltpu.VMEM((_B, _N), jnp.float32)],
    )(dist_pot_donors)
    return (idx, dist)


# speedup vs baseline: 2.6274x; 1.0012x over previous
import jax
import jax.numpy as jnp
from jax.experimental import pallas as pl
from jax.experimental.pallas import tpu as pltpu

_B = 128
_N = 8192
_K = 32


def _topk_body(x_ref, idx_ref, dist_ref, work_ref):
    cols = jax.lax.broadcasted_iota(jnp.int32, (_B, _N), 1)
    iprev = jnp.full((_B, 1), -1, jnp.int32)
    for k in range(_K):
        x = x_ref[:] if k == 0 else work_ref[:]
        y = jnp.where(cols == iprev, jnp.float32(jnp.inf), x)
        if k + 1 < _K:
            work_ref[:] = y
        m = jnp.min(y, axis=1, keepdims=True)
        idx = jnp.min(jnp.where(y == m, cols, jnp.int32(_N)), axis=1,
                      keepdims=True)
        dist_ref[:, k] = m[:, 0]
        idx_ref[:, k] = idx[:, 0]
        iprev = idx


def kernel(dist_pot_donors, n_neighbors):
    del n_neighbors
    idx, dist = pl.pallas_call(
        _topk_body,
        out_shape=[
            jax.ShapeDtypeStruct((_B, _K), jnp.int32),
            jax.ShapeDtypeStruct((_B, _K), jnp.float32),
        ],
        scratch_shapes=[pltpu.VMEM((_B, _N), jnp.float32)],
    )(dist_pot_donors)
    return (idx, dist)
